# baseline (speedup 1.0000x reference)
import functools

import jax
import jax.numpy as jnp
from jax import lax
from jax.experimental import pallas as pl
from jax.experimental.pallas import tpu as pltpu
from jax.experimental.pallas import tpu_sc as plsc

N = 10000
E = 320000
D_IN = 128
H1 = 128
H2 = 64

NC = 2
NS = 16
NW = NC * NS
CHUNK = 64
ET = 10112
NCHUNK = ET // CHUNK
NPAIR = NCHUNK // 2
EP = ET * NW
ZROWS = 128
ROWS_PER_TILE = 5 * ZROWS
NOUT = NS * ROWS_PER_TILE
ACC_ROWS = NOUT

_mesh = plsc.VectorSubcoreMesh(
    core_axis_name="c", subcore_axis_name="s", num_cores=NC, num_subcores=NS
)


def _zero_vmem(ref, rows, width):
    zv = jnp.zeros((16,), jnp.float32)
    lanes = width // 16

    def body(k, _):
        i = k // lanes
        j = k % lanes
        ref[i, pl.ds(j * 16, 16)] = zv
        return 0

    lax.fori_loop(0, rows * lanes, body, 0)


def _agg_body(width, g_hbm, src_hbm, dst_hbm, out_hbm,
              srcv, dstv, rows_a, rows_b, acc, gsem_a, gsem_b, ssem_a, ssem_b):
    c = lax.axis_index("c")
    s = lax.axis_index("s")
    t = (1 - c) * NS + s

    _zero_vmem(rows_a, CHUNK, width)
    base = s * ROWS_PER_TILE
    for i in range(ROWS_PER_TILE // CHUNK):
        pltpu.sync_copy(rows_a, acc.at[pl.ds(base + i * CHUNK, CHUNK)])

    pltpu.sync_copy(src_hbm.at[t], srcv)
    pltpu.sync_copy(dst_hbm.at[t], dstv)

    plsc.subcore_barrier()

    def g_start(j, buf, sem):
        pltpu.async_copy(g_hbm.at[srcv.at[j]], buf, sem)

    def g_wait(j, buf, sem):
        pltpu.make_async_copy(g_hbm.at[srcv.at[j]], buf, sem).wait()

    def s_start(j, buf, sem):
        pltpu.async_copy(buf, acc.at[dstv.at[j]], sem, add=True)

    def s_wait(j, buf, sem):
        pltpu.make_async_copy(buf, acc.at[dstv.at[j]], sem).wait()

    g_start(0, rows_a, gsem_a)
    g_wait(0, rows_a, gsem_a)
    g_start(1, rows_b, gsem_b)
    s_start(0, rows_a, ssem_a)
    g_wait(1, rows_b, gsem_b)
    s_wait(0, rows_a, ssem_a)
    g_start(2, rows_a, gsem_a)
    s_start(1, rows_b, ssem_b)

    def pair(q, _):
        j0 = 2 * q
        j1 = j0 + 1
        g_wait(j0, rows_a, gsem_a)
        s_wait(j1 - 2, rows_b, ssem_b)
        g_start(j1, rows_b, gsem_b)
        s_start(j0, rows_a, ssem_a)
        g_wait(j1, rows_b, gsem_b)
        s_wait(j0, rows_a, ssem_a)
        g_start(j0 + 2, rows_a, gsem_a)
        s_start(j1, rows_b, ssem_b)
        return 0

    lax.fori_loop(1, NPAIR - 1, pair, 0)

    j0 = NCHUNK - 2
    j1 = NCHUNK - 1
    g_wait(j0, rows_a, gsem_a)
    s_wait(j1 - 2, rows_b, ssem_b)
    g_start(j1, rows_b, gsem_b)
    s_start(j0, rows_a, ssem_a)
    g_wait(j1, rows_b, gsem_b)
    s_wait(j0, rows_a, ssem_a)
    s_start(j1, rows_b, ssem_b)
    s_wait(j1, rows_b, ssem_b)

    plsc.subcore_barrier()

    r0 = s * ROWS_PER_TILE
    pltpu.sync_copy(acc.at[pl.ds(r0, ROWS_PER_TILE)],
                    out_hbm.at[c].at[pl.ds(r0, ROWS_PER_TILE)])


def _make_agg(width):
    return pl.kernel(
        functools.partial(_agg_body, width),
        out_type=jax.ShapeDtypeStruct((NC, NOUT, width), jnp.float32),
        mesh=_mesh,
        scratch_types=[
            pltpu.VMEM((NCHUNK, CHUNK), jnp.int32),
            pltpu.VMEM((NCHUNK, CHUNK), jnp.int32),
            pltpu.VMEM((CHUNK, width), jnp.float32),
            pltpu.VMEM((CHUNK, width), jnp.float32),
            pltpu.VMEM_SHARED((ACC_ROWS, width), jnp.float32),
            pltpu.SemaphoreType.DMA,
            pltpu.SemaphoreType.DMA,
            pltpu.SemaphoreType.DMA,
            pltpu.SemaphoreType.DMA,
        ],
        compiler_params=pltpu.CompilerParams(use_tc_tiling_on_sc=False),
        name=f"gcn_agg_{width}",
    )


def _deg_body(dst_hbm, out_hbm, dstv, ones_b, zbuf, acc, sem):
    c = lax.axis_index("c")
    s = lax.axis_index("s")
    t = c * NS + s

    _zero_vmem(zbuf, ZROWS, 16)
    base = s * ROWS_PER_TILE
    for i in range(5):
        pltpu.sync_copy(zbuf, acc.at[pl.ds(base + i * ZROWS, ZROWS)])

    ov = jnp.ones((16,), jnp.float32)

    def fill(k, _):
        ones_b[k, pl.ds(0, 16)] = ov
        return 0

    lax.fori_loop(0, CHUNK, fill, 0)

    pltpu.sync_copy(dst_hbm.at[t], dstv)

    plsc.subcore_barrier()

    def chunk(j, _):
        pltpu.sync_copy(ones_b, acc.at[dstv.at[j]], add=True)
        return 0

    lax.fori_loop(0, NCHUNK, chunk, 0)

    plsc.subcore_barrier()

    r0 = s * ROWS_PER_TILE
    pltpu.sync_copy(acc.at[pl.ds(r0, ROWS_PER_TILE)],
                    out_hbm.at[c].at[pl.ds(r0, ROWS_PER_TILE)])


_deg_kernel = pl.kernel(
    _deg_body,
    out_type=jax.ShapeDtypeStruct((NC, NOUT, 16), jnp.float32),
    mesh=_mesh,
    scratch_types=[
        pltpu.VMEM((NCHUNK, CHUNK), jnp.int32),
        pltpu.VMEM((CHUNK, 16), jnp.float32),
        pltpu.VMEM((ZROWS, 16), jnp.float32),
        pltpu.VMEM_SHARED((ACC_ROWS, 16), jnp.float32),
        pltpu.SemaphoreType.DMA,
    ],
    compiler_params=pltpu.CompilerParams(use_tc_tiling_on_sc=False),
    name="gcn_deg",
)


BN = 1000


def _tc1_body(x_ref, w_ref, d0_ref, d1_ref, g_ref):
    deg = d0_ref[:, 0:1] + d1_ref[:, 0:1] + 1.0
    dinv = lax.rsqrt(deg)
    h = jnp.dot(x_ref[...], w_ref[...], preferred_element_type=jnp.float32)
    g_ref[...] = h * dinv


def _tc2_body(a0_ref, a1_ref, g1_ref, d0_ref, d1_ref, w_ref, b_ref, g2_ref):
    deg = d0_ref[:, 0:1] + d1_ref[:, 0:1] + 1.0
    dinv = lax.rsqrt(deg)
    h1 = jnp.maximum(dinv * (a0_ref[...] + a1_ref[...] + g1_ref[...])
                     + b_ref[...], 0.0)
    h2 = jnp.dot(h1, w_ref[...], preferred_element_type=jnp.float32)
    g2_ref[...] = h2 * dinv


def _tc3_body(a0_ref, a1_ref, g2_ref, d0_ref, d1_ref, w_ref, b_ref, out_ref):
    deg = d0_ref[:, 0:1] + d1_ref[:, 0:1] + 1.0
    dinv = lax.rsqrt(deg)
    h2 = jnp.maximum(dinv * (a0_ref[...] + a1_ref[...] + g2_ref[...])
                     + b_ref[...], 0.0)
    red = jnp.sum(h2 * w_ref[...], axis=1, keepdims=True)
    out_ref[...] = jnp.broadcast_to(red, out_ref.shape)


def _row_spec(width):
    return pl.BlockSpec((BN, width), lambda i: (i, 0))


def _full_spec(a, b):
    return pl.BlockSpec((a, b), lambda i: (0, 0))


def kernel(x, edge_index, W1, b1, W2, b2, Wfc, bfc):
    src = edge_index[0]
    dst = edge_index[1]
    pad = EP - E
    srcp = jnp.concatenate([src, jnp.zeros((pad,), jnp.int32)])
    dstp = jnp.concatenate([dst, jnp.full((pad,), N, jnp.int32)])
    srcp = srcp.reshape(NW, NCHUNK, CHUNK)
    dstp = dstp.reshape(NW, NCHUNK, CHUNK)

    degp = _deg_kernel(dstp)
    d0 = degp[0, :N, :8]
    d1 = degp[1, :N, :8]

    grid = (N // BN,)

    g1 = pl.pallas_call(
        _tc1_body,
        grid=grid,
        in_specs=[
            _row_spec(D_IN),
            _full_spec(D_IN, H1),
            _row_spec(8),
            _row_spec(8),
        ],
        out_specs=_row_spec(H1),
        out_shape=jax.ShapeDtypeStruct((N, H1), jnp.float32),
    )(x, W1, d0, d1)

    agg1 = _make_agg(H1)(g1, srcp, dstp)[:, :N]

    g2 = pl.pallas_call(
        _tc2_body,
        grid=grid,
        in_specs=[
            _row_spec(H1),
            _row_spec(H1),
            _row_spec(H1),
            _row_spec(8),
            _row_spec(8),
            _full_spec(H1, H2),
            _full_spec(1, H1),
        ],
        out_specs=_row_spec(H2),
        out_shape=jax.ShapeDtypeStruct((N, H2), jnp.float32),
    )(agg1[0], agg1[1], g1, d0, d1, W2, b1.reshape(1, H1))

    agg2 = _make_agg(H2)(g2, srcp, dstp)[:, :N]

    out = pl.pallas_call(
        _tc3_body,
        grid=grid,
        in_specs=[
            _row_spec(H2),
            _row_spec(H2),
            _row_spec(H2),
            _row_spec(8),
            _row_spec(8),
            _full_spec(1, H2),
            _full_spec(1, H2),
        ],
        out_specs=_row_spec(8),
        out_shape=jax.ShapeDtypeStruct((N, 8), jnp.float32),
    )(agg2[0], agg2[1], g2, d0, d1, Wfc.reshape(1, H2), b2.reshape(1, H2))

    return out[:, 0] + bfc[0]


# speedup vs baseline: 1.6680x; 1.6680x over previous
import functools

import jax
import jax.numpy as jnp
from jax import lax
from jax.experimental import pallas as pl
from jax.experimental.pallas import tpu as pltpu
from jax.experimental.pallas import tpu_sc as plsc

N = 10000
E = 320000
D_IN = 128
H1 = 128
H2 = 64

NC = 2
NS = 16
NW = NC * NS
CHUNK = 64
ET = 10112
NCHUNK = ET // CHUNK
NPAIR = NCHUNK // 2
EP = ET * NW
ZROWS = 128
ROWS_PER_TILE = 5 * ZROWS
NOUT = NS * ROWS_PER_TILE
ACC_ROWS = NOUT

_mesh = plsc.VectorSubcoreMesh(
    core_axis_name="c", subcore_axis_name="s", num_cores=NC, num_subcores=NS
)


def _zero_vmem(ref, rows, width):
    zv = jnp.zeros((16,), jnp.float32)
    lanes = width // 16

    def body(k, _):
        i = k // lanes
        j = k % lanes
        ref[i, pl.ds(j * 16, 16)] = zv
        return 0

    lax.fori_loop(0, rows * lanes, body, 0)


def _agg_body(width, g_hbm, src_hbm, dst_hbm, out_hbm,
              srcv, dstv, rows_a, rows_b, acc, gsem_a, gsem_b, ssem_a, ssem_b):
    c = lax.axis_index("c")
    s = lax.axis_index("s")
    t = c * NS + s

    _zero_vmem(rows_a, CHUNK, width)
    base = s * ROWS_PER_TILE
    for i in range(ROWS_PER_TILE // CHUNK):
        pltpu.sync_copy(rows_a, acc.at[pl.ds(base + i * CHUNK, CHUNK)])

    pltpu.sync_copy(src_hbm.at[t], srcv)
    pltpu.sync_copy(dst_hbm.at[t], dstv)

    plsc.subcore_barrier()

    def g_start(j, buf, sem):
        pltpu.async_copy(g_hbm.at[srcv.at[j]], buf, sem)

    def g_wait(j, buf, sem):
        pltpu.make_async_copy(g_hbm.at[srcv.at[j]], buf, sem).wait()

    def s_start(j, buf, sem):
        pltpu.async_copy(buf, acc.at[dstv.at[j]], sem, add=True)

    def s_wait(j, buf, sem):
        pltpu.make_async_copy(buf, acc.at[dstv.at[j]], sem).wait()

    g_start(0, rows_a, gsem_a)
    g_wait(0, rows_a, gsem_a)
    g_start(1, rows_b, gsem_b)
    s_start(0, rows_a, ssem_a)
    g_wait(1, rows_b, gsem_b)
    s_wait(0, rows_a, ssem_a)
    g_start(2, rows_a, gsem_a)
    s_start(1, rows_b, ssem_b)

    def pair(q, _):
        j0 = 2 * q
        j1 = j0 + 1
        g_wait(j0, rows_a, gsem_a)
        s_wait(j1 - 2, rows_b, ssem_b)
        g_start(j1, rows_b, gsem_b)
        s_start(j0, rows_a, ssem_a)
        g_wait(j1, rows_b, gsem_b)
        s_wait(j0, rows_a, ssem_a)
        g_start(j0 + 2, rows_a, gsem_a)
        s_start(j1, rows_b, ssem_b)
        return 0

    lax.fori_loop(1, NPAIR - 1, pair, 0)

    j0 = NCHUNK - 2
    j1 = NCHUNK - 1
    g_wait(j0, rows_a, gsem_a)
    s_wait(j1 - 2, rows_b, ssem_b)
    g_start(j1, rows_b, gsem_b)
    s_start(j0, rows_a, ssem_a)
    g_wait(j1, rows_b, gsem_b)
    s_wait(j0, rows_a, ssem_a)
    s_start(j1, rows_b, ssem_b)
    s_wait(j1, rows_b, ssem_b)

    plsc.subcore_barrier()

    r0 = s * ROWS_PER_TILE
    pltpu.sync_copy(acc.at[pl.ds(r0, ROWS_PER_TILE)],
                    out_hbm.at[c].at[pl.ds(r0, ROWS_PER_TILE)])


def _make_agg(width):
    return pl.kernel(
        functools.partial(_agg_body, width),
        out_type=jax.ShapeDtypeStruct((NC, NOUT, width), jnp.float32),
        mesh=_mesh,
        scratch_types=[
            pltpu.VMEM((NCHUNK, CHUNK), jnp.int32),
            pltpu.VMEM((NCHUNK, CHUNK), jnp.int32),
            pltpu.VMEM((CHUNK, width), jnp.float32),
            pltpu.VMEM((CHUNK, width), jnp.float32),
            pltpu.VMEM_SHARED((ACC_ROWS, width), jnp.float32),
            pltpu.SemaphoreType.DMA,
            pltpu.SemaphoreType.DMA,
            pltpu.SemaphoreType.DMA,
            pltpu.SemaphoreType.DMA,
        ],
        compiler_params=pltpu.CompilerParams(use_tc_tiling_on_sc=False),
        name=f"gcn_agg_{width}",
    )


def _deg_body(dst_hbm, out_hbm, dstv, ones_b, zbuf, acc, sem):
    c = lax.axis_index("c")
    s = lax.axis_index("s")
    t = c * NS + s

    _zero_vmem(zbuf, ZROWS, 16)
    base = s * ROWS_PER_TILE
    for i in range(5):
        pltpu.sync_copy(zbuf, acc.at[pl.ds(base + i * ZROWS, ZROWS)])

    ov = jnp.ones((16,), jnp.float32)

    def fill(k, _):
        ones_b[k, pl.ds(0, 16)] = ov
        return 0

    lax.fori_loop(0, CHUNK, fill, 0)

    pltpu.sync_copy(dst_hbm.at[t], dstv)

    plsc.subcore_barrier()

    def chunk(j, _):
        pltpu.sync_copy(ones_b, acc.at[dstv.at[j]], add=True)
        return 0

    lax.fori_loop(0, NCHUNK, chunk, 0)

    plsc.subcore_barrier()

    r0 = s * ROWS_PER_TILE
    pltpu.sync_copy(acc.at[pl.ds(r0, ROWS_PER_TILE)],
                    out_hbm.at[c].at[pl.ds(r0, ROWS_PER_TILE)])


_deg_kernel = pl.kernel(
    _deg_body,
    out_type=jax.ShapeDtypeStruct((NC, NOUT, 16), jnp.float32),
    mesh=_mesh,
    scratch_types=[
        pltpu.VMEM((NCHUNK, CHUNK), jnp.int32),
        pltpu.VMEM((CHUNK, 16), jnp.float32),
        pltpu.VMEM((ZROWS, 16), jnp.float32),
        pltpu.VMEM_SHARED((ACC_ROWS, 16), jnp.float32),
        pltpu.SemaphoreType.DMA,
    ],
    compiler_params=pltpu.CompilerParams(use_tc_tiling_on_sc=False),
    name="gcn_deg",
)


BN = 1000


def _tc1_body(x_ref, w_ref, d0_ref, d1_ref, g_ref):
    deg = d0_ref[:, 0:1] + d1_ref[:, 0:1] + 1.0
    dinv = lax.rsqrt(deg)
    h = jnp.dot(x_ref[...], w_ref[...], preferred_element_type=jnp.float32)
    g_ref[...] = h * dinv


def _tc2_body(a0_ref, a1_ref, g1_ref, d0_ref, d1_ref, w_ref, b_ref, g2_ref):
    deg = d0_ref[:, 0:1] + d1_ref[:, 0:1] + 1.0
    dinv = lax.rsqrt(deg)
    h1 = jnp.maximum(dinv * (a0_ref[...] + a1_ref[...] + g1_ref[...])
                     + b_ref[...], 0.0)
    h2 = jnp.dot(h1, w_ref[...], preferred_element_type=jnp.float32)
    g2_ref[...] = h2 * dinv


def _tc3_body(a0_ref, a1_ref, g2_ref, d0_ref, d1_ref, w_ref, b_ref, out_ref):
    deg = d0_ref[:, 0:1] + d1_ref[:, 0:1] + 1.0
    dinv = lax.rsqrt(deg)
    h2 = jnp.maximum(dinv * (a0_ref[...] + a1_ref[...] + g2_ref[...])
                     + b_ref[...], 0.0)
    red = jnp.sum(h2 * w_ref[...], axis=1, keepdims=True)
    out_ref[...] = jnp.broadcast_to(red, out_ref.shape)


def _row_spec(width):
    return pl.BlockSpec((BN, width), lambda i: (i, 0))


def _full_spec(a, b):
    return pl.BlockSpec((a, b), lambda i: (0, 0))


def kernel(x, edge_index, W1, b1, W2, b2, Wfc, bfc):
    src = edge_index[0]
    dst = edge_index[1]
    pad = EP - E
    pidx = jnp.arange(pad, dtype=jnp.int32)
    srcp = jnp.concatenate([src, pidx % 64])
    dstp = jnp.concatenate([dst, N + (pidx % 128)])
    srcp = srcp.reshape(NW, NCHUNK, CHUNK)
    dstp = dstp.reshape(NW, NCHUNK, CHUNK)

    degp = _deg_kernel(dstp)
    d0 = degp[0, :N, :8]
    d1 = degp[1, :N, :8]

    grid = (N // BN,)

    g1 = pl.pallas_call(
        _tc1_body,
        grid=grid,
        in_specs=[
            _row_spec(D_IN),
            _full_spec(D_IN, H1),
            _row_spec(8),
            _row_spec(8),
        ],
        out_specs=_row_spec(H1),
        out_shape=jax.ShapeDtypeStruct((N, H1), jnp.float32),
    )(x, W1, d0, d1)

    agg1 = _make_agg(H1)(g1, srcp, dstp)[:, :N]

    g2 = pl.pallas_call(
        _tc2_body,
        grid=grid,
        in_specs=[
            _row_spec(H1),
            _row_spec(H1),
            _row_spec(H1),
            _row_spec(8),
            _row_spec(8),
            _full_spec(H1, H2),
            _full_spec(1, H1),
        ],
        out_specs=_row_spec(H2),
        out_shape=jax.ShapeDtypeStruct((N, H2), jnp.float32),
    )(agg1[0], agg1[1], g1, d0, d1, W2, b1.reshape(1, H1))

    agg2 = _make_agg(H2)(g2, srcp, dstp)[:, :N]

    out = pl.pallas_call(
        _tc3_body,
        grid=grid,
        in_specs=[
            _row_spec(H2),
            _row_spec(H2),
            _row_spec(H2),
            _row_spec(8),
            _row_spec(8),
            _full_spec(1, H2),
            _full_spec(1, H2),
        ],
        out_specs=_row_spec(8),
        out_shape=jax.ShapeDtypeStruct((N, 8), jnp.float32),
    )(agg2[0], agg2[1], g2, d0, d1, Wfc.reshape(1, H2), b2.reshape(1, H2))

    return out[:, 0] + bfc[0]


# speedup vs baseline: 2.3261x; 1.3946x over previous
import functools

import jax
import jax.numpy as jnp
from jax import lax
from jax.experimental import pallas as pl
from jax.experimental.pallas import tpu as pltpu
from jax.experimental.pallas import tpu_sc as plsc

N = 10000
E = 320000
D_IN = 128
H1 = 128
H2 = 64

NC = 2
NS = 16
NW = NC * NS
CHUNK = 64
ET = 10112
NCHUNK = ET // CHUNK
EP = ET * NW
ZROWS = 128
ROWS_PER_TILE = 5 * ZROWS
NOUT = NS * ROWS_PER_TILE
ACC_ROWS = NOUT

_mesh = plsc.VectorSubcoreMesh(
    core_axis_name="c", subcore_axis_name="s", num_cores=NC, num_subcores=NS
)


def _zero_vmem(ref, rows, width):
    zv = jnp.zeros((16,), jnp.float32)
    lanes = width // 16

    def body(k, _):
        i = k // lanes
        j = k % lanes
        ref[i, pl.ds(j * 16, 16)] = zv
        return 0

    lax.fori_loop(0, rows * lanes, body, 0)


def _agg_body(width, g_hbm, src_hbm, dst_hbm, out_hbm,
              srcv, dstv, rows_a, rows_b, rows_c, acc,
              gsem_a, gsem_b, gsem_c, ssem_a, ssem_b, ssem_c):
    c = lax.axis_index("c")
    s = lax.axis_index("s")
    t = c * NS + s

    _zero_vmem(rows_a, CHUNK, width)
    base = s * ROWS_PER_TILE
    for i in range(ROWS_PER_TILE // CHUNK):
        pltpu.sync_copy(rows_a, acc.at[pl.ds(base + i * CHUNK, CHUNK)])

    pltpu.sync_copy(src_hbm.at[t], srcv)
    pltpu.sync_copy(dst_hbm.at[t], dstv)

    plsc.subcore_barrier()

    bufs = (rows_a, rows_b, rows_c)
    gsems = (gsem_a, gsem_b, gsem_c)
    ssems = (ssem_a, ssem_b, ssem_c)

    def g_start(j, b):
        pltpu.async_copy(g_hbm.at[srcv.at[j]], bufs[b], gsems[b])

    def g_wait(j, b):
        pltpu.make_async_copy(g_hbm.at[srcv.at[j]], bufs[b], gsems[b]).wait()

    def s_start(j, b):
        pltpu.async_copy(bufs[b], acc.at[dstv.at[j]], ssems[b], add=True)

    def s_wait(j, b):
        pltpu.make_async_copy(bufs[b], acc.at[dstv.at[j]], ssems[b]).wait()

    def step(j, b):
        g_wait(j, b)
        s_start(j, b)
        s_wait(j - 1, (b + 2) % 3)
        g_start(j + 2, (b + 2) % 3)

    g_start(0, 0)
    g_start(1, 1)
    g_wait(0, 0)
    s_start(0, 0)
    g_start(2, 2)
    step(1, 1)
    step(2, 2)

    def tri(q, _):
        j = 3 * q
        step(j, 0)
        step(j + 1, 1)
        step(j + 2, 2)
        return 0

    lax.fori_loop(1, (NCHUNK - 2) // 3, tri, 0)

    j0 = NCHUNK - 2
    j1 = NCHUNK - 1
    b0 = j0 % 3
    b1 = j1 % 3
    g_wait(j0, b0)
    s_start(j0, b0)
    s_wait(j0 - 1, (b0 + 2) % 3)
    g_wait(j1, b1)
    s_start(j1, b1)
    s_wait(j0, b0)
    s_wait(j1, b1)

    plsc.subcore_barrier()

    r0 = s * ROWS_PER_TILE
    pltpu.sync_copy(acc.at[pl.ds(r0, ROWS_PER_TILE)],
                    out_hbm.at[c].at[pl.ds(r0, ROWS_PER_TILE)])


def _make_agg(width):
    return pl.kernel(
        functools.partial(_agg_body, width),
        out_type=jax.ShapeDtypeStruct((NC, NOUT, width), jnp.float32),
        mesh=_mesh,
        scratch_types=[
            pltpu.VMEM((NCHUNK, CHUNK), jnp.int32),
            pltpu.VMEM((NCHUNK, CHUNK), jnp.int32),
            pltpu.VMEM((CHUNK, width), jnp.float32),
            pltpu.VMEM((CHUNK, width), jnp.float32),
            pltpu.VMEM((CHUNK, width), jnp.float32),
            pltpu.VMEM_SHARED((ACC_ROWS, width), jnp.float32),
            pltpu.SemaphoreType.DMA,
            pltpu.SemaphoreType.DMA,
            pltpu.SemaphoreType.DMA,
            pltpu.SemaphoreType.DMA,
            pltpu.SemaphoreType.DMA,
            pltpu.SemaphoreType.DMA,
        ],
        compiler_params=pltpu.CompilerParams(use_tc_tiling_on_sc=False),
        name=f"gcn_agg_{width}",
    )


def _deg_body(dst_hbm, out_hbm, dstv, ones_b, zbuf, acc, sem):
    c = lax.axis_index("c")
    s = lax.axis_index("s")
    t = c * NS + s

    _zero_vmem(zbuf, ZROWS, 16)
    base = s * ROWS_PER_TILE
    for i in range(5):
        pltpu.sync_copy(zbuf, acc.at[pl.ds(base + i * ZROWS, ZROWS)])

    ov = jnp.ones((16,), jnp.float32)

    def fill(k, _):
        ones_b[k, pl.ds(0, 16)] = ov
        return 0

    lax.fori_loop(0, CHUNK, fill, 0)

    pltpu.sync_copy(dst_hbm.at[t], dstv)

    plsc.subcore_barrier()

    def chunk(j, _):
        pltpu.sync_copy(ones_b, acc.at[dstv.at[j]], add=True)
        return 0

    lax.fori_loop(0, NCHUNK, chunk, 0)

    plsc.subcore_barrier()

    r0 = s * ROWS_PER_TILE
    pltpu.sync_copy(acc.at[pl.ds(r0, ROWS_PER_TILE)],
                    out_hbm.at[c].at[pl.ds(r0, ROWS_PER_TILE)])


_deg_kernel = pl.kernel(
    _deg_body,
    out_type=jax.ShapeDtypeStruct((NC, NOUT, 16), jnp.float32),
    mesh=_mesh,
    scratch_types=[
        pltpu.VMEM((NCHUNK, CHUNK), jnp.int32),
        pltpu.VMEM((CHUNK, 16), jnp.float32),
        pltpu.VMEM((ZROWS, 16), jnp.float32),
        pltpu.VMEM_SHARED((ACC_ROWS, 16), jnp.float32),
        pltpu.SemaphoreType.DMA,
    ],
    compiler_params=pltpu.CompilerParams(use_tc_tiling_on_sc=False),
    name="gcn_deg",
)


BN = 1000


def _tc1_body(x_ref, w_ref, d0_ref, d1_ref, g_ref):
    deg = d0_ref[:, 0:1] + d1_ref[:, 0:1] + 1.0
    dinv = lax.rsqrt(deg)
    h = jnp.dot(x_ref[...], w_ref[...], preferred_element_type=jnp.float32)
    g_ref[...] = h * dinv


def _tc2_body(a0_ref, a1_ref, g1_ref, d0_ref, d1_ref, w_ref, b_ref, g2_ref):
    deg = d0_ref[:, 0:1] + d1_ref[:, 0:1] + 1.0
    dinv = lax.rsqrt(deg)
    h1 = jnp.maximum(dinv * (a0_ref[...] + a1_ref[...] + g1_ref[...])
                     + b_ref[...], 0.0)
    h2 = jnp.dot(h1, w_ref[...], preferred_element_type=jnp.float32)
    g2_ref[...] = h2 * dinv


def _tc3_body(a0_ref, a1_ref, g2_ref, d0_ref, d1_ref, w_ref, b_ref, out_ref):
    deg = d0_ref[:, 0:1] + d1_ref[:, 0:1] + 1.0
    dinv = lax.rsqrt(deg)
    h2 = jnp.maximum(dinv * (a0_ref[...] + a1_ref[...] + g2_ref[...])
                     + b_ref[...], 0.0)
    red = jnp.sum(h2 * w_ref[...], axis=1, keepdims=True)
    out_ref[...] = jnp.broadcast_to(red, out_ref.shape)


def _row_spec(width):
    return pl.BlockSpec((BN, width), lambda i: (i, 0))


def _full_spec(a, b):
    return pl.BlockSpec((a, b), lambda i: (0, 0))


def kernel(x, edge_index, W1, b1, W2, b2, Wfc, bfc):
    src = edge_index[0]
    dst = edge_index[1]
    pad = EP - E
    pidx = jnp.arange(pad, dtype=jnp.int32)
    srcp = jnp.concatenate([src, pidx % 64])
    dstp = jnp.concatenate([dst, N + (pidx % 128)])
    srcp = srcp.reshape(NW, NCHUNK, CHUNK)
    dstp = dstp.reshape(NW, NCHUNK, CHUNK)

    degp = _deg_kernel(dstp)
    d0 = degp[0, :N, :8]
    d1 = degp[1, :N, :8]

    grid = (N // BN,)

    g1 = pl.pallas_call(
        _tc1_body,
        grid=grid,
        in_specs=[
            _row_spec(D_IN),
            _full_spec(D_IN, H1),
            _row_spec(8),
            _row_spec(8),
        ],
        out_specs=_row_spec(H1),
        out_shape=jax.ShapeDtypeStruct((N, H1), jnp.float32),
    )(x, W1, d0, d1)

    agg1 = _make_agg(H1)(g1, srcp, dstp)[:, :N]

    g2 = pl.pallas_call(
        _tc2_body,
        grid=grid,
        in_specs=[
            _row_spec(H1),
            _row_spec(H1),
            _row_spec(H1),
            _row_spec(8),
            _row_spec(8),
            _full_spec(H1, H2),
            _full_spec(1, H1),
        ],
        out_specs=_row_spec(H2),
        out_shape=jax.ShapeDtypeStruct((N, H2), jnp.float32),
    )(agg1[0], agg1[1], g1, d0, d1, W2, b1.reshape(1, H1))

    agg2 = _make_agg(H2)(g2, srcp, dstp)[:, :N]

    out = pl.pallas_call(
        _tc3_body,
        grid=grid,
        in_specs=[
            _row_spec(H2),
            _row_spec(H2),
            _row_spec(H2),
            _row_spec(8),
            _row_spec(8),
            _full_spec(1, H2),
            _full_spec(1, H2),
        ],
        out_specs=_row_spec(8),
        out_shape=jax.ShapeDtypeStruct((N, 8), jnp.float32),
    )(agg2[0], agg2[1], g2, d0, d1, Wfc.reshape(1, H2), b2.reshape(1, H2))

    return out[:, 0] + bfc[0]


# speedup vs baseline: 2.4246x; 1.0423x over previous
import functools

import jax
import jax.numpy as jnp
from jax import lax
from jax.experimental import pallas as pl
from jax.experimental.pallas import tpu as pltpu
from jax.experimental.pallas import tpu_sc as plsc

N = 10000
E = 320000
D_IN = 128
H1 = 128
H2 = 64

NC = 2
NS = 16
NW = NC * NS
CHUNK = 32
ET = 10112
NCHUNK = ET // CHUNK
KBUF = 6
EP = ET * NW
ZROWS = 128
ROWS_PER_TILE = 5 * ZROWS
NOUT = NS * ROWS_PER_TILE
ACC_ROWS = NOUT

_mesh = plsc.VectorSubcoreMesh(
    core_axis_name="c", subcore_axis_name="s", num_cores=NC, num_subcores=NS
)


def _zero_vmem(ref, rows, width):
    zv = jnp.zeros((16,), jnp.float32)
    lanes = width // 16

    def body(k, _):
        i = k // lanes
        j = k % lanes
        ref[i, pl.ds(j * 16, 16)] = zv
        return 0

    lax.fori_loop(0, rows * lanes, body, 0)


def _agg_body(width, g_hbm, src_hbm, dst_hbm, out_hbm, *scratch):
    srcv, dstv = scratch[0], scratch[1]
    bufs = scratch[2:2 + KBUF]
    acc = scratch[2 + KBUF]
    gsems = scratch[3 + KBUF:3 + 2 * KBUF]
    ssems = scratch[3 + 2 * KBUF:3 + 3 * KBUF]

    c = lax.axis_index("c")
    s = lax.axis_index("s")
    t = c * NS + s

    _zero_vmem(bufs[0], CHUNK, width)
    base = s * ROWS_PER_TILE
    for i in range(ROWS_PER_TILE // CHUNK):
        pltpu.sync_copy(bufs[0], acc.at[pl.ds(base + i * CHUNK, CHUNK)])

    pltpu.sync_copy(src_hbm.at[t], srcv)
    pltpu.sync_copy(dst_hbm.at[t], dstv)

    plsc.subcore_barrier()

    def g_start(j, b):
        pltpu.async_copy(g_hbm.at[srcv.at[j]], bufs[b], gsems[b])

    def g_wait(j, b):
        pltpu.make_async_copy(g_hbm.at[srcv.at[j]], bufs[b], gsems[b]).wait()

    def s_start(j, b):
        pltpu.async_copy(bufs[b], acc.at[dstv.at[j]], ssems[b], add=True)

    def s_wait(j, b):
        pltpu.make_async_copy(bufs[b], acc.at[dstv.at[j]], ssems[b]).wait()

    def step(j, b, fetch=True):
        g_wait(j, b)
        s_start(j, b)
        s_wait(j - 1, (b + KBUF - 1) % KBUF)
        if fetch:
            g_start(j + KBUF - 1, (b + KBUF - 1) % KBUF)

    for b in range(KBUF - 1):
        g_start(b, b)
    g_wait(0, 0)
    s_start(0, 0)
    g_start(KBUF - 1, KBUF - 1)
    for j in range(1, KBUF):
        step(j, j % KBUF)

    QHI = (NCHUNK - KBUF) // KBUF

    def blk(q, _):
        j0 = KBUF * q
        for b in range(KBUF):
            step(j0 + b, b)
        return 0

    lax.fori_loop(1, QHI, blk, 0)

    for j in range(KBUF * QHI, NCHUNK):
        step(j, j % KBUF, fetch=(j + KBUF - 1 < NCHUNK))
    s_wait(NCHUNK - 1, (NCHUNK - 1) % KBUF)

    plsc.subcore_barrier()

    r0 = s * ROWS_PER_TILE
    pltpu.sync_copy(acc.at[pl.ds(r0, ROWS_PER_TILE)],
                    out_hbm.at[c].at[pl.ds(r0, ROWS_PER_TILE)])


def _make_agg(width):
    scratch = [
        pltpu.VMEM((NCHUNK, CHUNK), jnp.int32),
        pltpu.VMEM((NCHUNK, CHUNK), jnp.int32),
    ]
    scratch += [pltpu.VMEM((CHUNK, width), jnp.float32)] * KBUF
    scratch += [pltpu.VMEM_SHARED((ACC_ROWS, width), jnp.float32)]
    scratch += [pltpu.SemaphoreType.DMA] * (2 * KBUF)
    return pl.kernel(
        functools.partial(_agg_body, width),
        out_type=jax.ShapeDtypeStruct((NC, NOUT, width), jnp.float32),
        mesh=_mesh,
        scratch_types=scratch,
        compiler_params=pltpu.CompilerParams(use_tc_tiling_on_sc=False),
        name=f"gcn_agg_{width}",
    )


def _deg_body(dst_hbm, out_hbm, dstv, ones_b, zbuf, acc, sem):
    c = lax.axis_index("c")
    s = lax.axis_index("s")
    t = c * NS + s

    _zero_vmem(zbuf, ZROWS, 16)
    base = s * ROWS_PER_TILE
    for i in range(5):
        pltpu.sync_copy(zbuf, acc.at[pl.ds(base + i * ZROWS, ZROWS)])

    ov = jnp.ones((16,), jnp.float32)

    def fill(k, _):
        ones_b[k, pl.ds(0, 16)] = ov
        return 0

    lax.fori_loop(0, CHUNK, fill, 0)

    pltpu.sync_copy(dst_hbm.at[t], dstv)

    plsc.subcore_barrier()

    def chunk(j, _):
        pltpu.sync_copy(ones_b, acc.at[dstv.at[j]], add=True)
        return 0

    lax.fori_loop(0, NCHUNK, chunk, 0)

    plsc.subcore_barrier()

    r0 = s * ROWS_PER_TILE
    pltpu.sync_copy(acc.at[pl.ds(r0, ROWS_PER_TILE)],
                    out_hbm.at[c].at[pl.ds(r0, ROWS_PER_TILE)])


_deg_kernel = pl.kernel(
    _deg_body,
    out_type=jax.ShapeDtypeStruct((NC, NOUT, 16), jnp.float32),
    mesh=_mesh,
    scratch_types=[
        pltpu.VMEM((NCHUNK, CHUNK), jnp.int32),
        pltpu.VMEM((CHUNK, 16), jnp.float32),
        pltpu.VMEM((ZROWS, 16), jnp.float32),
        pltpu.VMEM_SHARED((ACC_ROWS, 16), jnp.float32),
        pltpu.SemaphoreType.DMA,
    ],
    compiler_params=pltpu.CompilerParams(use_tc_tiling_on_sc=False),
    name="gcn_deg",
)


BN = 1000


def _tc1_body(x_ref, w_ref, d0_ref, d1_ref, g_ref):
    deg = d0_ref[:, 0:1] + d1_ref[:, 0:1] + 1.0
    dinv = lax.rsqrt(deg)
    h = jnp.dot(x_ref[...], w_ref[...], preferred_element_type=jnp.float32)
    g_ref[...] = h * dinv


def _tc2_body(a0_ref, a1_ref, g1_ref, d0_ref, d1_ref, w_ref, b_ref, g2_ref):
    deg = d0_ref[:, 0:1] + d1_ref[:, 0:1] + 1.0
    dinv = lax.rsqrt(deg)
    h1 = jnp.maximum(dinv * (a0_ref[...] + a1_ref[...] + g1_ref[...])
                     + b_ref[...], 0.0)
    h2 = jnp.dot(h1, w_ref[...], preferred_element_type=jnp.float32)
    g2_ref[...] = h2 * dinv


def _tc3_body(a0_ref, a1_ref, g2_ref, d0_ref, d1_ref, w_ref, b_ref, out_ref):
    deg = d0_ref[:, 0:1] + d1_ref[:, 0:1] + 1.0
    dinv = lax.rsqrt(deg)
    h2 = jnp.maximum(dinv * (a0_ref[...] + a1_ref[...] + g2_ref[...])
                     + b_ref[...], 0.0)
    red = jnp.sum(h2 * w_ref[...], axis=1, keepdims=True)
    out_ref[...] = jnp.broadcast_to(red, out_ref.shape)


def _row_spec(width):
    return pl.BlockSpec((BN, width), lambda i: (i, 0))


def _full_spec(a, b):
    return pl.BlockSpec((a, b), lambda i: (0, 0))


def kernel(x, edge_index, W1, b1, W2, b2, Wfc, bfc):
    src = edge_index[0]
    dst = edge_index[1]
    pad = EP - E
    pidx = jnp.arange(pad, dtype=jnp.int32)
    srcp = jnp.concatenate([src, pidx % 64])
    dstp = jnp.concatenate([dst, N + (pidx % 128)])
    srcp = srcp.reshape(NW, NCHUNK, CHUNK)
    dstp = dstp.reshape(NW, NCHUNK, CHUNK)

    degp = _deg_kernel(dstp)
    d0 = degp[0, :N, :8]
    d1 = degp[1, :N, :8]

    grid = (N // BN,)

    g1 = pl.pallas_call(
        _tc1_body,
        grid=grid,
        in_specs=[
            _row_spec(D_IN),
            _full_spec(D_IN, H1),
            _row_spec(8),
            _row_spec(8),
        ],
        out_specs=_row_spec(H1),
        out_shape=jax.ShapeDtypeStruct((N, H1), jnp.float32),
    )(x, W1, d0, d1)

    agg1 = _make_agg(H1)(g1, srcp, dstp)[:, :N]

    g2 = pl.pallas_call(
        _tc2_body,
        grid=grid,
        in_specs=[
            _row_spec(H1),
            _row_spec(H1),
            _row_spec(H1),
            _row_spec(8),
            _row_spec(8),
            _full_spec(H1, H2),
            _full_spec(1, H1),
        ],
        out_specs=_row_spec(H2),
        out_shape=jax.ShapeDtypeStruct((N, H2), jnp.float32),
    )(agg1[0], agg1[1], g1, d0, d1, W2, b1.reshape(1, H1))

    agg2 = _make_agg(H2)(g2, srcp, dstp)[:, :N]

    out = pl.pallas_call(
        _tc3_body,
        grid=grid,
        in_specs=[
            _row_spec(H2),
            _row_spec(H2),
            _row_spec(H2),
            _row_spec(8),
            _row_spec(8),
            _full_spec(1, H2),
            _full_spec(1, H2),
        ],
        out_specs=_row_spec(8),
        out_shape=jax.ShapeDtypeStruct((N, 8), jnp.float32),
    )(agg2[0], agg2[1], g2, d0, d1, Wfc.reshape(1, H2), b2.reshape(1, H2))

    return out[:, 0] + bfc[0]


# speedup vs baseline: 2.7863x; 1.1492x over previous
import functools

import jax
import jax.numpy as jnp
from jax import lax
from jax.experimental import pallas as pl
from jax.experimental.pallas import tpu as pltpu
from jax.experimental.pallas import tpu_sc as plsc

N = 10000
E = 320000
D_IN = 128
H1 = 128
H2 = 64

NC = 2
NS = 16
NW = NC * NS
CHUNK = 40
ET = E // NW
NCHUNK = ET // CHUNK
KBUF = 5
ROWS_PER_TILE = 640
NOUT = NS * ROWS_PER_TILE
ACC_ROWS = NOUT

_mesh = plsc.VectorSubcoreMesh(
    core_axis_name="c", subcore_axis_name="s", num_cores=NC, num_subcores=NS
)


def _zero_vmem(ref, rows, width):
    zv = jnp.zeros((16,), jnp.float32)
    lanes = width // 16

    def body(k, _):
        i = k // lanes
        j = k % lanes
        ref[i, pl.ds(j * 16, 16)] = zv
        return 0

    lax.fori_loop(0, rows * lanes, body, 0)


def _agg_body(width, g_hbm, src_hbm, dst_hbm, out_hbm, *scratch):
    srcv, dstv = scratch[0], scratch[1]
    bufs = scratch[2:2 + KBUF]
    acc = scratch[2 + KBUF]
    gsems = scratch[3 + KBUF:3 + 2 * KBUF]
    ssems = scratch[3 + 2 * KBUF:3 + 3 * KBUF]

    c = lax.axis_index("c")
    s = lax.axis_index("s")
    t = c * NS + s

    _zero_vmem(bufs[0], CHUNK, width)
    base = s * ROWS_PER_TILE
    for i in range(ROWS_PER_TILE // CHUNK):
        pltpu.sync_copy(bufs[0], acc.at[pl.ds(base + i * CHUNK, CHUNK)])

    pltpu.sync_copy(src_hbm.at[t], srcv)
    pltpu.sync_copy(dst_hbm.at[t], dstv)

    plsc.subcore_barrier()

    def g_start(j, b):
        pltpu.async_copy(g_hbm.at[srcv.at[j]], bufs[b], gsems[b])

    def g_wait(j, b):
        pltpu.make_async_copy(g_hbm.at[srcv.at[j]], bufs[b], gsems[b]).wait()

    def s_start(j, b):
        pltpu.async_copy(bufs[b], acc.at[dstv.at[j]], ssems[b], add=True)

    def s_wait(j, b):
        pltpu.make_async_copy(bufs[b], acc.at[dstv.at[j]], ssems[b]).wait()

    def step(j, b, fetch=True):
        g_wait(j, b)
        s_start(j, b)
        s_wait(j - 1, (b + KBUF - 1) % KBUF)
        if fetch:
            g_start(j + KBUF - 1, (b + KBUF - 1) % KBUF)

    for b in range(KBUF - 1):
        g_start(b, b)
    g_wait(0, 0)
    s_start(0, 0)
    g_start(KBUF - 1, KBUF - 1)
    for j in range(1, KBUF):
        step(j, j % KBUF)

    QHI = (NCHUNK - KBUF) // KBUF

    def blk(q, _):
        j0 = KBUF * q
        for b in range(KBUF):
            step(j0 + b, b)
        return 0

    lax.fori_loop(1, QHI, blk, 0)

    for j in range(KBUF * QHI, NCHUNK):
        step(j, j % KBUF, fetch=(j + KBUF - 1 < NCHUNK))
    s_wait(NCHUNK - 1, (NCHUNK - 1) % KBUF)

    plsc.subcore_barrier()

    r0 = s * ROWS_PER_TILE
    pltpu.sync_copy(acc.at[pl.ds(r0, ROWS_PER_TILE)],
                    out_hbm.at[c].at[pl.ds(r0, ROWS_PER_TILE)])


def _make_agg(width):
    scratch = [
        pltpu.VMEM((NCHUNK, CHUNK), jnp.int32),
        pltpu.VMEM((NCHUNK, CHUNK), jnp.int32),
    ]
    scratch += [pltpu.VMEM((CHUNK, width), jnp.float32)] * KBUF
    scratch += [pltpu.VMEM_SHARED((ACC_ROWS, width), jnp.float32)]
    scratch += [pltpu.SemaphoreType.DMA] * (2 * KBUF)
    return pl.kernel(
        functools.partial(_agg_body, width),
        out_type=jax.ShapeDtypeStruct((NC, NOUT, width), jnp.float32),
        mesh=_mesh,
        scratch_types=scratch,
        compiler_params=pltpu.CompilerParams(use_tc_tiling_on_sc=False),
        name=f"gcn_agg_{width}",
    )


def _deg_body(dst_hbm, out_hbm, dstv, ones_b, zbuf, acc, sem):
    c = lax.axis_index("c")
    s = lax.axis_index("s")
    t = c * NS + s

    _zero_vmem(zbuf, 128, 16)
    base = s * ROWS_PER_TILE
    for i in range(ROWS_PER_TILE // 128):
        pltpu.sync_copy(zbuf, acc.at[pl.ds(base + i * 128, 128)])

    ov = jnp.ones((16,), jnp.float32)

    def fill(k, _):
        ones_b[k, pl.ds(0, 16)] = ov
        return 0

    lax.fori_loop(0, CHUNK, fill, 0)

    pltpu.sync_copy(dst_hbm.at[t], dstv)

    plsc.subcore_barrier()

    W = 4

    def fire(j):
        pltpu.async_copy(ones_b, acc.at[dstv.at[j]], sem, add=True)

    def drain(j):
        pltpu.make_async_copy(ones_b, acc.at[dstv.at[j]], sem).wait()

    for j in range(W):
        fire(j)

    def chunk(j, _):
        fire(j)
        drain(j - W)
        return 0

    lax.fori_loop(W, NCHUNK, chunk, 0)
    for j in range(NCHUNK - W, NCHUNK):
        drain(j)

    plsc.subcore_barrier()

    r0 = s * ROWS_PER_TILE
    pltpu.sync_copy(acc.at[pl.ds(r0, ROWS_PER_TILE)],
                    out_hbm.at[c].at[pl.ds(r0, ROWS_PER_TILE)])


_deg_kernel = pl.kernel(
    _deg_body,
    out_type=jax.ShapeDtypeStruct((NC, NOUT, 16), jnp.float32),
    mesh=_mesh,
    scratch_types=[
        pltpu.VMEM((NCHUNK, CHUNK), jnp.int32),
        pltpu.VMEM((CHUNK, 16), jnp.float32),
        pltpu.VMEM((128, 16), jnp.float32),
        pltpu.VMEM_SHARED((ACC_ROWS, 16), jnp.float32),
        pltpu.SemaphoreType.DMA,
    ],
    compiler_params=pltpu.CompilerParams(use_tc_tiling_on_sc=False),
    name="gcn_deg",
)


BN = 1000


def _dinv(d0_ref, d1_ref):
    deg = d0_ref[0, :, 0:1] + d1_ref[0, :, 0:1] + 1.0
    return lax.rsqrt(deg)


def _tc1_body(x_ref, w_ref, d0_ref, d1_ref, g_ref):
    dinv = _dinv(d0_ref, d1_ref)
    h = jnp.dot(x_ref[...], w_ref[...], preferred_element_type=jnp.float32)
    g_ref[...] = h * dinv


def _tc2_body(a0_ref, a1_ref, g1_ref, d0_ref, d1_ref, w_ref, b_ref, g2_ref):
    dinv = _dinv(d0_ref, d1_ref)
    h1 = jnp.maximum(dinv * (a0_ref[0] + a1_ref[0] + g1_ref[...])
                     + b_ref[...], 0.0)
    h2 = jnp.dot(h1, w_ref[...], preferred_element_type=jnp.float32)
    g2_ref[...] = h2 * dinv


def _tc3_body(a0_ref, a1_ref, g2_ref, d0_ref, d1_ref, w_ref, b_ref, out_ref):
    dinv = _dinv(d0_ref, d1_ref)
    h2 = jnp.maximum(dinv * (a0_ref[0] + a1_ref[0] + g2_ref[...])
                     + b_ref[...], 0.0)
    red = jnp.sum(h2 * w_ref[...], axis=1, keepdims=True)
    out_ref[...] = jnp.broadcast_to(red, out_ref.shape)


def _row_spec(width):
    return pl.BlockSpec((BN, width), lambda i: (i, 0))


def _full_spec(a, b):
    return pl.BlockSpec((a, b), lambda i: (0, 0))


def kernel(x, edge_index, W1, b1, W2, b2, Wfc, bfc):
    srcp = edge_index[0].reshape(NW, NCHUNK, CHUNK)
    dstp = edge_index[1].reshape(NW, NCHUNK, CHUNK)

    degp = _deg_kernel(dstp)

    grid = (N // BN,)

    def deg_spec(c):
        return pl.BlockSpec((1, BN, 16), lambda i, c=c: (c, i, 0))

    def agg_spec(c, width):
        return pl.BlockSpec((1, BN, width), lambda i, c=c: (c, i, 0))

    g1 = pl.pallas_call(
        _tc1_body,
        grid=grid,
        in_specs=[
            _row_spec(D_IN),
            _full_spec(D_IN, H1),
            deg_spec(0),
            deg_spec(1),
        ],
        out_specs=_row_spec(H1),
        out_shape=jax.ShapeDtypeStruct((N, H1), jnp.float32),
    )(x, W1, degp, degp)

    agg1 = _make_agg(H1)(g1, srcp, dstp)

    g2 = pl.pallas_call(
        _tc2_body,
        grid=grid,
        in_specs=[
            agg_spec(0, H1),
            agg_spec(1, H1),
            _row_spec(H1),
            deg_spec(0),
            deg_spec(1),
            _full_spec(H1, H2),
            _full_spec(1, H1),
        ],
        out_specs=_row_spec(H2),
        out_shape=jax.ShapeDtypeStruct((N, H2), jnp.float32),
    )(agg1, agg1, g1, degp, degp, W2, b1.reshape(1, H1))

    agg2 = _make_agg(H2)(g2, srcp, dstp)

    out = pl.pallas_call(
        _tc3_body,
        grid=grid,
        in_specs=[
            agg_spec(0, H2),
            agg_spec(1, H2),
            _row_spec(H2),
            deg_spec(0),
            deg_spec(1),
            _full_spec(1, H2),
            _full_spec(1, H2),
        ],
        out_specs=_row_spec(8),
        out_shape=jax.ShapeDtypeStruct((N, 8), jnp.float32),
    )(agg2, agg2, g2, degp, degp, Wfc.reshape(1, H2), b2.reshape(1, H2))

    return out[:, 0] + bfc[0]
